# Initial kernel scaffold; baseline (speedup 1.0000x reference)
import jax
import jax.numpy as jnp
from jax.experimental import pallas as pl


def kernel(seg4, edge4, w_s1, b_s1, w_s11, b_s11, w_mlp, b_mlp, w_s2, b_s2, w_s3, b_s3, w_g2, b_g2):
    raise NotImplementedError("write your pallas kernel here")


# speedup vs baseline: 1.2451x; 1.2451x over previous
import jax
import jax.numpy as jnp
from jax.experimental import pallas as pl
from jax.experimental.pallas import tpu as pltpu


def _raw_view_hw_s(p, hw, s):
    nblk = 2 * hw // s
    na = s // 2
    p3 = p.reshape(na, 2, hw)
    pe = p3[:, 0, :]
    po = p3[:, 1, :]
    slabs = []
    for m in range(nblk):
        start = m * s
        if start + s <= hw:
            slabs.append(pe[:, start:start + s])
        elif start >= hw:
            off = start - hw
            slabs.append(po[:, off:off + s])
        else:
            head = hw - start
            slabs.append(jnp.concatenate(
                [pe[:, start:], po[:, :s - head]], axis=1))
    stacked = jnp.stack(slabs, axis=1)
    return stacked.reshape(hw, s)


def _fused_kernel(seg_ref, edge_ref, wst_ref, bst_ref, wmlp_ref, bmlp_ref,
                  wg2_ref, bg2_ref, sc_ref, out_ref):
    seg = seg_ref[0]
    c, hw = seg.shape
    s = wmlp_ref.shape[0]

    proj = jnp.dot(wst_ref[...], seg,
                   preferred_element_type=jnp.float32) + bst_ref[...]
    p1 = proj[:s]
    p2 = proj[s:]

    theta = _raw_view_hw_s(p1, hw, s)
    sigma_t = _raw_view_hw_s(p2, hw, s)

    mean_c = jnp.sum(seg, axis=1, keepdims=True) * (1.0 / hw)
    ca = jnp.maximum(
        jnp.dot(wmlp_ref[...], mean_c, preferred_element_type=jnp.float32)
        + bmlp_ref[...], 0.0)
    v = jnp.dot(theta, ca, preferred_element_type=jnp.float32)
    u = jnp.sum(ca * p1, axis=0, keepdims=True)
    pre_c = v * u
    m1 = jnp.max(pre_c, axis=0, keepdims=True)
    e1 = jnp.exp(pre_c - m1)
    sim = e1 * pl.reciprocal(jnp.sum(e1, axis=0, keepdims=True), approx=True)

    sigma_out = jnp.dot(sigma_t, p2,
                        preferred_element_type=jnp.float32)
    w2 = sc_ref[0]
    b2 = sc_ref[1]
    w3 = sc_ref[2]
    b3 = sc_ref[3]
    erow = jnp.max(seg + edge_ref[0], axis=0, keepdims=True)
    seg_ss = w2 * jnp.max(seg, axis=0, keepdims=True) + b2
    edge_mm = w3 * jnp.transpose(erow) + b3
    dsa = (edge_mm * seg_ss) * sigma_out
    m2 = jnp.max(dsa, axis=0, keepdims=True)
    e2 = jnp.exp(dsa - m2)
    sim = sim + e2 * pl.reciprocal(jnp.sum(e2, axis=0, keepdims=True),
                                   approx=True)

    seg_similar = jnp.dot(seg, sim,
                          preferred_element_type=jnp.float32)
    gout = jnp.maximum(
        jnp.dot(wg2_ref[...], seg_similar, preferred_element_type=jnp.float32)
        + bg2_ref[...], 0.0)
    out_ref[0] = gout + seg + seg


def kernel(seg4, edge4, w_s1, b_s1, w_s11, b_s11, w_mlp, b_mlp,
           w_s2, b_s2, w_s3, b_s3, w_g2, b_g2):
    n, c, h, w = seg4.shape
    hw = h * w
    num_s = w_s1.shape[0]

    seg = seg4.reshape(n, c, hw).astype(jnp.float32)
    edge = edge4.reshape(n, c, hw).astype(jnp.float32)

    w_stack = jnp.concatenate([w_s1, w_s11], axis=0)
    b_stack = jnp.concatenate([b_s1, b_s11], axis=0)
    scalars = jnp.stack([w_s2, b_s2, w_s3, b_s3]).astype(jnp.float32)

    def full(shape):
        nd = len(shape)
        return pl.BlockSpec(shape, lambda b, _nd=nd: (0,) * _nd)

    def batched(shape):
        nd = len(shape)
        return pl.BlockSpec((1,) + shape, lambda b, _nd=nd: (b,) + (0,) * _nd)

    out = pl.pallas_call(
        _fused_kernel,
        grid=(n,),
        in_specs=[batched((c, hw)), batched((c, hw)),
                  full((2 * num_s, c)), full((2 * num_s, 1)),
                  full((num_s, c)), full((num_s, 1)),
                  full((c, c)), full((c, 1)),
                  pl.BlockSpec(memory_space=pltpu.MemorySpace.SMEM)],
        out_specs=batched((c, hw)),
        out_shape=jax.ShapeDtypeStruct((n, c, hw), jnp.float32),
        compiler_params=pltpu.CompilerParams(dimension_semantics=("parallel",)),
    )(seg, edge, w_stack, b_stack, w_mlp, b_mlp, w_g2, b_g2, scalars)

    return out.reshape(n, c, h, w)


# speedup vs baseline: 1.4638x; 1.1756x over previous
import numpy as np

import jax
import jax.numpy as jnp
from jax.experimental import pallas as pl
from jax.experimental.pallas import tpu as pltpu


def _slab_view_hw_s(p, hw, s):
    nblk = 2 * hw // s
    na = s // 2
    p3 = p.reshape(na, 2, hw)
    pe = p3[:, 0, :]
    po = p3[:, 1, :]
    slabs = []
    for m in range(nblk):
        start = m * s
        if start + s <= hw:
            slabs.append(pe[:, start:start + s])
        elif start >= hw:
            off = start - hw
            slabs.append(po[:, off:off + s])
        else:
            head = hw - start
            slabs.append(jnp.concatenate(
                [pe[:, start:], po[:, :s - head]], axis=1))
    return jnp.concatenate(slabs, axis=0)


def _row_perm(hw, s):
    nblk = 2 * hw // s
    na = s // 2
    ip = np.arange(hw)
    return nblk * (ip % na) + ip // na


def _fused_kernel(seg_ref, edge_ref, wst_ref, bst_ref, wmlp_ref, bmlp_ref,
                  wg2_ref, bg2_ref, perm_ref, sc_ref, out_ref):
    seg = seg_ref[0]
    c, hw = seg.shape
    s = wmlp_ref.shape[0]

    proj = jnp.dot(wst_ref[...], seg,
                   preferred_element_type=jnp.float32) + bst_ref[...]
    p1 = proj[:s]
    p2 = proj[s:]

    theta = _slab_view_hw_s(p1, hw, s)
    sigma_t = _slab_view_hw_s(p2, hw, s)

    mean_c = jnp.sum(seg, axis=1, keepdims=True) * (1.0 / hw)
    ca = jnp.maximum(
        jnp.dot(wmlp_ref[...], mean_c, preferred_element_type=jnp.float32)
        + bmlp_ref[...], 0.0)
    v = jnp.dot(theta, ca, preferred_element_type=jnp.float32)
    u = jnp.sum(ca * p1, axis=0, keepdims=True)
    e1 = jnp.exp(v * u)
    sim = e1 * pl.reciprocal(jnp.sum(e1, axis=0, keepdims=True), approx=True)

    sigma_out = jnp.dot(sigma_t, p2,
                        preferred_element_type=jnp.float32)
    w2 = sc_ref[0]
    b2 = sc_ref[1]
    w3 = sc_ref[2]
    b3 = sc_ref[3]
    erow = jnp.max(seg + edge_ref[0], axis=0, keepdims=True)
    erow_p = jnp.dot(erow, perm_ref[...],
                     preferred_element_type=jnp.float32)
    seg_ss = w2 * jnp.max(seg, axis=0, keepdims=True) + b2
    edge_mm = w3 * jnp.transpose(erow_p) + b3
    e2 = jnp.exp((edge_mm * seg_ss) * sigma_out)
    sim = sim + e2 * pl.reciprocal(jnp.sum(e2, axis=0, keepdims=True),
                                   approx=True)

    seg_p = jnp.dot(seg, perm_ref[...],
                    preferred_element_type=jnp.float32)
    seg_similar = jnp.dot(seg_p, sim,
                          preferred_element_type=jnp.float32)
    gout = jnp.maximum(
        jnp.dot(wg2_ref[...], seg_similar, preferred_element_type=jnp.float32)
        + bg2_ref[...], 0.0)
    out_ref[0] = gout + seg + seg


def kernel(seg4, edge4, w_s1, b_s1, w_s11, b_s11, w_mlp, b_mlp,
           w_s2, b_s2, w_s3, b_s3, w_g2, b_g2):
    n, c, h, w = seg4.shape
    hw = h * w
    num_s = w_s1.shape[0]

    seg = seg4.reshape(n, c, hw).astype(jnp.float32)
    edge = edge4.reshape(n, c, hw).astype(jnp.float32)

    w_stack = jnp.concatenate([w_s1, w_s11], axis=0)
    b_stack = jnp.concatenate([b_s1, b_s11], axis=0)
    scalars = jnp.stack([w_s2, b_s2, w_s3, b_s3]).astype(jnp.float32)

    pi = _row_perm(hw, num_s)
    perm_np = np.zeros((hw, hw), np.float32)
    perm_np[pi, np.arange(hw)] = 1.0
    perm = jnp.asarray(perm_np)

    def full(shape):
        nd = len(shape)
        return pl.BlockSpec(shape, lambda b, _nd=nd: (0,) * _nd)

    def batched(shape):
        nd = len(shape)
        return pl.BlockSpec((1,) + shape, lambda b, _nd=nd: (b,) + (0,) * _nd)

    out = pl.pallas_call(
        _fused_kernel,
        grid=(n,),
        in_specs=[batched((c, hw)), batched((c, hw)),
                  full((2 * num_s, c)), full((2 * num_s, 1)),
                  full((num_s, c)), full((num_s, 1)),
                  full((c, c)), full((c, 1)),
                  full((hw, hw)),
                  pl.BlockSpec(memory_space=pltpu.MemorySpace.SMEM)],
        out_specs=batched((c, hw)),
        out_shape=jax.ShapeDtypeStruct((n, c, hw), jnp.float32),
        compiler_params=pltpu.CompilerParams(dimension_semantics=("parallel",)),
    )(seg, edge, w_stack, b_stack, w_mlp, b_mlp, w_g2, b_g2, perm, scalars)

    return out.reshape(n, c, h, w)
